# baseline (speedup 1.0000x reference)
import jax
import jax.numpy as jnp
from jax import lax
from jax.experimental import pallas as pl
from jax.experimental.pallas import tpu as pltpu
from jax.experimental.pallas import tpu_sc as plsc

BATCH = 4
SEQ_LEN = 8192
D_MODEL = 1024

NUM_CORES = 2
NUM_SUBCORES = 16
NUM_WORKERS = NUM_CORES * NUM_SUBCORES
ROWS_PER_WORKER = SEQ_LEN // NUM_WORKERS
CHUNK_ROWS = 8
NUM_CHUNKS = ROWS_PER_WORKER // CHUNK_ROWS
NUM_PAIRS = BATCH // 2
NUM_STEPS = NUM_CHUNKS * NUM_PAIRS
LANES = 16
GROUPS_PER_ROW = D_MODEL // LANES
GROUPS_PER_CHUNK = CHUNK_ROWS * GROUPS_PER_ROW


def _sc_body(x_hbm, pe_hbm, out_hbm, lset0, lset1, sset0, sset1,
             ls0, ls1, ss0, ss1):
    wid = lax.axis_index("s") * NUM_CORES + lax.axis_index("c")
    base = wid * ROWS_PER_WORKER

    load_sets = (lset0, lset1)
    store_sets = (sset0, sset1)
    ld_sems = (ls0, ls1)
    st_sems = (ss0, ss1)

    def step_coords(c):
        j, p = divmod(c, NUM_PAIRS)
        return base + j * CHUNK_ROWS, 2 * p

    def issue_loads(c, i):
        row, b = step_coords(c)
        bufs = load_sets[i]
        return [
            pltpu.async_copy(
                pe_hbm.at[pl.ds(row, CHUNK_ROWS)], bufs[0], ld_sems[i]
            ),
            pltpu.async_copy(
                x_hbm.at[b, pl.ds(row, CHUNK_ROWS)], bufs[1], ld_sems[i]
            ),
            pltpu.async_copy(
                x_hbm.at[b + 1, pl.ds(row, CHUNK_ROWS)], bufs[2], ld_sems[i]
            ),
        ]

    ld_descs = [None, None]
    st_descs = [None, None]

    ld_descs[0] = issue_loads(0, 0)
    ld_descs[1] = issue_loads(1, 1)

    for c in range(NUM_STEPS):
        i = c % 2
        row, b = step_coords(c)
        for d in ld_descs[i]:
            d.wait()
        if st_descs[i] is not None:
            for d in st_descs[i]:
                d.wait()

        pe_buf, xb0, xb1 = load_sets[i]
        yb0, yb1 = store_sets[i]

        @plsc.parallel_loop(0, GROUPS_PER_CHUNK, unroll=4)
        def _(g):
            r = g >> 6
            k = (g & (GROUPS_PER_ROW - 1)) * LANES
            sl = pl.ds(k, LANES)
            pe = pe_buf[r, sl]
            yb0[r, sl] = xb0[r, sl] + pe
            yb1[r, sl] = xb1[r, sl] + pe

        st_descs[i] = [
            pltpu.async_copy(
                yb0, out_hbm.at[b, pl.ds(row, CHUNK_ROWS)], st_sems[i]
            ),
            pltpu.async_copy(
                yb1, out_hbm.at[b + 1, pl.ds(row, CHUNK_ROWS)], st_sems[i]
            ),
        ]
        if c + 2 < NUM_STEPS:
            ld_descs[i] = issue_loads(c + 2, i)

    for descs in st_descs:
        if descs is not None:
            for d in descs:
                d.wait()


def kernel(x, pos_embed):
    mesh = plsc.VectorSubcoreMesh(
        core_axis_name="c", subcore_axis_name="s",
        num_cores=NUM_CORES, num_subcores=NUM_SUBCORES,
    )
    load_set = [
        pltpu.VMEM((CHUNK_ROWS, D_MODEL), jnp.float32) for _ in range(3)
    ]
    store_set = [
        pltpu.VMEM((CHUNK_ROWS, D_MODEL), jnp.float32) for _ in range(2)
    ]
    return pl.kernel(
        _sc_body,
        out_type=jax.ShapeDtypeStruct((BATCH, SEQ_LEN, D_MODEL), jnp.float32),
        mesh=mesh,
        scratch_types=[
            load_set,
            load_set,
            store_set,
            store_set,
            pltpu.SemaphoreType.DMA,
            pltpu.SemaphoreType.DMA,
            pltpu.SemaphoreType.DMA,
            pltpu.SemaphoreType.DMA,
        ],
    )(x, pos_embed)


# speedup vs baseline: 1.0924x; 1.0924x over previous
import jax
import jax.numpy as jnp
from jax import lax
from jax.experimental import pallas as pl
from jax.experimental.pallas import tpu as pltpu
from jax.experimental.pallas import tpu_sc as plsc

BATCH = 4
SEQ_LEN = 8192
D_MODEL = 1024

NUM_CORES = 2
NUM_SUBCORES = 16
NUM_WORKERS = NUM_CORES * NUM_SUBCORES
ROWS_PER_WORKER = SEQ_LEN // NUM_WORKERS
CHUNK_ROWS = 4
NUM_STEPS = ROWS_PER_WORKER // CHUNK_ROWS
LANES = 16
GROUPS_PER_ROW = D_MODEL // LANES
GROUPS_PER_CHUNK = CHUNK_ROWS * GROUPS_PER_ROW
NBUF = 3


def _sc_body(x_hbm, pe_hbm, out_hbm, l0, l1, l2, s0, s1, s2,
             ls0, ls1, ls2, ss0, ss1, ss2):
    wid = lax.axis_index("s") * NUM_CORES + lax.axis_index("c")
    base = wid * ROWS_PER_WORKER

    load_sets = (l0, l1, l2)
    store_sets = (s0, s1, s2)
    ld_sems = (ls0, ls1, ls2)
    st_sems = (ss0, ss1, ss2)

    def issue_loads(c, i):
        row = base + c * CHUNK_ROWS
        bufs = load_sets[i]
        descs = [
            pltpu.async_copy(
                pe_hbm.at[pl.ds(row, CHUNK_ROWS)], bufs[0], ld_sems[i]
            )
        ]
        for b in range(BATCH):
            descs.append(
                pltpu.async_copy(
                    x_hbm.at[b, pl.ds(row, CHUNK_ROWS)], bufs[1 + b], ld_sems[i]
                )
            )
        return descs

    ld_descs = [None] * NBUF
    st_descs = [None] * NBUF

    for i in range(NBUF):
        ld_descs[i] = issue_loads(i, i)

    for c in range(NUM_STEPS):
        i = c % NBUF
        row = base + c * CHUNK_ROWS
        for d in ld_descs[i]:
            d.wait()
        if st_descs[i] is not None:
            for d in st_descs[i]:
                d.wait()

        pe_buf, xb0, xb1, xb2, xb3 = load_sets[i]
        yb0, yb1, yb2, yb3 = store_sets[i]

        @plsc.parallel_loop(0, GROUPS_PER_CHUNK, unroll=4)
        def _(g):
            r = g >> 6
            k = (g & (GROUPS_PER_ROW - 1)) * LANES
            sl = pl.ds(k, LANES)
            pe = pe_buf[r, sl]
            yb0[r, sl] = xb0[r, sl] + pe
            yb1[r, sl] = xb1[r, sl] + pe
            yb2[r, sl] = xb2[r, sl] + pe
            yb3[r, sl] = xb3[r, sl] + pe

        ybufs = (yb0, yb1, yb2, yb3)
        st_descs[i] = [
            pltpu.async_copy(
                ybufs[b], out_hbm.at[b, pl.ds(row, CHUNK_ROWS)], st_sems[i]
            )
            for b in range(BATCH)
        ]
        if c + NBUF < NUM_STEPS:
            ld_descs[i] = issue_loads(c + NBUF, i)

    for descs in st_descs:
        if descs is not None:
            for d in descs:
                d.wait()


def kernel(x, pos_embed):
    mesh = plsc.VectorSubcoreMesh(
        core_axis_name="c", subcore_axis_name="s",
        num_cores=NUM_CORES, num_subcores=NUM_SUBCORES,
    )
    load_set = [
        pltpu.VMEM((CHUNK_ROWS, D_MODEL), jnp.float32) for _ in range(1 + BATCH)
    ]
    store_set = [
        pltpu.VMEM((CHUNK_ROWS, D_MODEL), jnp.float32) for _ in range(BATCH)
    ]
    return pl.kernel(
        _sc_body,
        out_type=jax.ShapeDtypeStruct((BATCH, SEQ_LEN, D_MODEL), jnp.float32),
        mesh=mesh,
        scratch_types=[
            load_set,
            load_set,
            load_set,
            store_set,
            store_set,
            store_set,
            pltpu.SemaphoreType.DMA,
            pltpu.SemaphoreType.DMA,
            pltpu.SemaphoreType.DMA,
            pltpu.SemaphoreType.DMA,
            pltpu.SemaphoreType.DMA,
            pltpu.SemaphoreType.DMA,
        ],
    )(x, pos_embed)


# speedup vs baseline: 1.1267x; 1.0314x over previous
import jax
import jax.numpy as jnp
from jax import lax
from jax.experimental import pallas as pl
from jax.experimental.pallas import tpu as pltpu
from jax.experimental.pallas import tpu_sc as plsc

BATCH = 4
SEQ_LEN = 8192
D_MODEL = 1024

NUM_CORES = 2
NUM_SUBCORES = 16
NUM_WORKERS = NUM_CORES * NUM_SUBCORES
ROWS_PER_WORKER = SEQ_LEN // NUM_WORKERS
CHUNK_ROWS = 8
NUM_STEPS = ROWS_PER_WORKER // CHUNK_ROWS
LANES = 16
GROUPS_PER_ROW = D_MODEL // LANES
GROUPS_PER_CHUNK = CHUNK_ROWS * GROUPS_PER_ROW


NBUF = 3


def _sc_body(x_hbm, pe_hbm, out_hbm, bufs0, bufs1, bufs2,
             ls0, ls1, ls2, ss0, ss1, ss2):
    wid = lax.axis_index("s") * NUM_CORES + lax.axis_index("c")
    base = wid * ROWS_PER_WORKER

    buf_sets = (bufs0, bufs1, bufs2)
    ld_sems = (ls0, ls1, ls2)
    st_sems = (ss0, ss1, ss2)

    def issue_loads(c, i):
        row = base + c * CHUNK_ROWS
        bufs = buf_sets[i]
        descs = [
            pltpu.async_copy(
                pe_hbm.at[pl.ds(row, CHUNK_ROWS)], bufs[0], ld_sems[i]
            )
        ]
        for b in range(BATCH):
            descs.append(
                pltpu.async_copy(
                    x_hbm.at[b, pl.ds(row, CHUNK_ROWS)], bufs[1 + b], ld_sems[i]
                )
            )
        return descs

    ld_descs = [None] * NBUF
    st_descs = [None] * NBUF

    ld_descs[0] = issue_loads(0, 0)
    ld_descs[1] = issue_loads(1, 1)

    for c in range(NUM_STEPS):
        i = c % NBUF
        row = base + c * CHUNK_ROWS
        bufs = buf_sets[i]
        if c + 2 < NUM_STEPS:
            ni = (c + 2) % NBUF
            if st_descs[ni] is not None:
                for d in st_descs[ni]:
                    d.wait()
            ld_descs[ni] = issue_loads(c + 2, ni)
        for d in ld_descs[i]:
            d.wait()

        pe_buf, xb0, xb1, xb2, xb3 = bufs

        @plsc.parallel_loop(0, GROUPS_PER_CHUNK, unroll=4)
        def _(g):
            r = g >> 6
            k = (g & (GROUPS_PER_ROW - 1)) * LANES
            sl = pl.ds(k, LANES)
            pe = pe_buf[r, sl]
            xb0[r, sl] = xb0[r, sl] + pe
            xb1[r, sl] = xb1[r, sl] + pe
            xb2[r, sl] = xb2[r, sl] + pe
            xb3[r, sl] = xb3[r, sl] + pe

        st_descs[i] = [
            pltpu.async_copy(
                bufs[1 + b], out_hbm.at[b, pl.ds(row, CHUNK_ROWS)], st_sems[i]
            )
            for b in range(BATCH)
        ]

    for descs in st_descs:
        if descs is not None:
            for d in descs:
                d.wait()


def kernel(x, pos_embed):
    mesh = plsc.VectorSubcoreMesh(
        core_axis_name="c", subcore_axis_name="s",
        num_cores=NUM_CORES, num_subcores=NUM_SUBCORES,
    )
    buf_set = [
        pltpu.VMEM((CHUNK_ROWS, D_MODEL), jnp.float32) for _ in range(1 + BATCH)
    ]
    return pl.kernel(
        _sc_body,
        out_type=jax.ShapeDtypeStruct((BATCH, SEQ_LEN, D_MODEL), jnp.float32),
        mesh=mesh,
        scratch_types=[
            buf_set,
            buf_set,
            buf_set,
            pltpu.SemaphoreType.DMA,
            pltpu.SemaphoreType.DMA,
            pltpu.SemaphoreType.DMA,
            pltpu.SemaphoreType.DMA,
            pltpu.SemaphoreType.DMA,
            pltpu.SemaphoreType.DMA,
        ],
    )(x, pos_embed)
